# Initial kernel scaffold; baseline (speedup 1.0000x reference)
import jax
import jax.numpy as jnp
from jax.experimental import pallas as pl


def kernel(X, edge_list, W0, b0, W1, b1, W2, b2, W3, b3):
    raise NotImplementedError("write your pallas kernel here")


# speedup vs baseline: 19.5232x; 19.5232x over previous
import functools

import jax
import jax.numpy as jnp
from jax import lax
from jax.experimental import pallas as pl
from jax.experimental.pallas import tpu as pltpu
from jax.experimental.pallas import tpu_sc as plsc

NC = 2
NS = 16
NW = NC * NS
LANES = 16

N = 10000
NPAD = 10240
E = 320000
D = 128
DH = D // 2
HID = 128
C = 40

EPT = E // NW
CW = 125
NCH = EPT // CW
RPT = NPAD // NS
ZR = 64


@functools.cache
def _mesh():
    return plsc.VectorSubcoreMesh(
        core_axis_name="c", subcore_axis_name="s",
        num_cores=NC, num_subcores=NS)


def _wid():
    return lax.axis_index("s") * NC + lax.axis_index("c")


def _sc_hist_body(dst_hbm, out_hbm, idx_v, hist_v):
    wid = _wid()
    pltpu.sync_copy(dst_hbm.at[pl.ds(wid * EPT, EPT)], idx_v)
    zeros = jnp.zeros((LANES,), jnp.float32)

    def zbody(i, c):
        hist_v[pl.ds(i * LANES, LANES)] = zeros
        return c

    lax.fori_loop(0, NPAD // LANES, zbody, 0)
    ones = jnp.ones((LANES,), jnp.float32)

    def body(i, c):
        idx = idx_v[pl.ds(i * LANES, LANES)]
        plsc.addupdate_scatter(hist_v, [idx], ones)
        return c

    lax.fori_loop(0, EPT // LANES, body, 0)
    pltpu.sync_copy(hist_v, out_hbm.at[wid])


@functools.cache
def _sc_hist():
    return pl.kernel(
        _sc_hist_body,
        out_type=jax.ShapeDtypeStruct((NW, NPAD), jnp.float32),
        mesh=_mesh(),
        scratch_types=[
            pltpu.VMEM((EPT,), jnp.int32),
            pltpu.VMEM((NPAD,), jnp.float32),
        ],
        compiler_params=pltpu.CompilerParams(needs_layout_passes=False),
    )


def _sc_s_body(src_hbm, dst_hbm, invs_hbm, out_hbm, src_v, dst_v, invs_v, s_v):
    wid = _wid()
    pltpu.sync_copy(src_hbm.at[pl.ds(wid * EPT, EPT)], src_v)
    pltpu.sync_copy(dst_hbm.at[pl.ds(wid * EPT, EPT)], dst_v)
    pltpu.sync_copy(invs_hbm, invs_v)
    zeros = jnp.zeros((LANES,), jnp.float32)

    def zbody(i, c):
        s_v[pl.ds(i * LANES, LANES)] = zeros
        return c

    lax.fori_loop(0, NPAD // LANES, zbody, 0)

    def body(i, c):
        d16 = dst_v[pl.ds(i * LANES, LANES)]
        s16 = src_v[pl.ds(i * LANES, LANES)]
        vals = plsc.load_gather(invs_v, [d16])
        plsc.addupdate_scatter(s_v, [s16], vals)
        return c

    lax.fori_loop(0, EPT // LANES, body, 0)
    pltpu.sync_copy(s_v, out_hbm.at[wid])


@functools.cache
def _sc_s():
    return pl.kernel(
        _sc_s_body,
        out_type=jax.ShapeDtypeStruct((NW, NPAD), jnp.float32),
        mesh=_mesh(),
        scratch_types=[
            pltpu.VMEM((EPT,), jnp.int32),
            pltpu.VMEM((EPT,), jnp.int32),
            pltpu.VMEM((NPAD,), jnp.float32),
            pltpu.VMEM((NPAD,), jnp.float32),
        ],
        compiler_params=pltpu.CompilerParams(needs_layout_passes=False),
    )


def _sc_agg_body(src_hbm, dst_hbm, hs_lo_hbm, hs_hi_hbm, out_hbm,
                 src_v, dst_v, rows_a, rows_b, zbuf, agg_sh, sem_a, sem_b):
    cid = lax.axis_index("c")
    sid = lax.axis_index("s")
    wid = sid * NC + cid
    pltpu.sync_copy(src_hbm.at[pl.ds(wid * NCH, NCH)], src_v)
    pltpu.sync_copy(dst_hbm.at[pl.ds(wid * NCH, NCH)], dst_v)

    zeros = jnp.zeros((LANES,), jnp.float32)

    def zb(i, c):
        r = i // (DH // LANES)
        col = (i % (DH // LANES)) * LANES
        zbuf[r, pl.ds(col, LANES)] = zeros
        return c

    lax.fori_loop(0, ZR * DH // LANES, zb, 0)

    for half, hs_hbm in ((0, hs_lo_hbm), (1, hs_hi_hbm)):
        def zcopy(t, c):
            pltpu.sync_copy(zbuf, agg_sh.at[pl.ds(sid * RPT + t * ZR, ZR)])
            return c

        lax.fori_loop(0, RPT // ZR, zcopy, 0)
        plsc.subcore_barrier()

        pltpu.async_copy(hs_hbm.at[src_v.at[0]], rows_a, sem_a)

        def body(i, c):
            j = i * 2
            pltpu.async_copy(hs_hbm.at[src_v.at[j + 1]], rows_b, sem_b)
            pltpu.make_async_copy(hs_hbm.at[src_v.at[j]], rows_a, sem_a).wait()
            pltpu.sync_copy(rows_a, agg_sh.at[dst_v.at[j]], add=True)

            @pl.when(i < NCH // 2 - 1)
            def _():
                pltpu.async_copy(hs_hbm.at[src_v.at[j + 2]], rows_a, sem_a)

            pltpu.make_async_copy(
                hs_hbm.at[src_v.at[j + 1]], rows_b, sem_b).wait()
            pltpu.sync_copy(rows_b, agg_sh.at[dst_v.at[j + 1]], add=True)
            return c

        lax.fori_loop(0, NCH // 2, body, 0)
        plsc.subcore_barrier()
        pltpu.sync_copy(
            agg_sh.at[pl.ds(sid * RPT, RPT)],
            out_hbm.at[half].at[cid].at[pl.ds(sid * RPT, RPT)])


@functools.cache
def _sc_agg():
    return pl.kernel(
        _sc_agg_body,
        out_type=jax.ShapeDtypeStruct((2, NC, NPAD, DH), jnp.float32),
        mesh=_mesh(),
        scratch_types=[
            pltpu.VMEM((NCH, CW), jnp.int32),
            pltpu.VMEM((NCH, CW), jnp.int32),
            pltpu.VMEM((CW, DH), jnp.float32),
            pltpu.VMEM((CW, DH), jnp.float32),
            pltpu.VMEM((ZR, DH), jnp.float32),
            pltpu.VMEM_SHARED((NPAD, DH), jnp.float32),
            pltpu.SemaphoreType.DMA,
            pltpu.SemaphoreType.DMA,
        ],
        compiler_params=pltpu.CompilerParams(
            needs_layout_passes=False, use_tc_tiling_on_sc=False),
    )


_R = 1024


def _tc_prep_body(hist_ref, x_ref, invs_ref, hs_lo_ref, hs_hi_ref):
    deg = 1.0 + jnp.sum(hist_ref[...], axis=0)
    invs = lax.rsqrt(deg)
    invs_ref[...] = invs[:, None]
    hs = x_ref[...] * invs[:, None]
    hs_lo_ref[...] = hs[:, :DH]
    hs_hi_ref[...] = hs[:, DH:]


def _tc_prep(hist, x_pad):
    return pl.pallas_call(
        _tc_prep_body,
        grid=(NPAD // _R,),
        in_specs=[
            pl.BlockSpec((NW, _R), lambda i: (0, i)),
            pl.BlockSpec((_R, D), lambda i: (i, 0)),
        ],
        out_specs=[
            pl.BlockSpec((_R, 1), lambda i: (i, 0)),
            pl.BlockSpec((_R, DH), lambda i: (i, 0)),
            pl.BlockSpec((_R, DH), lambda i: (i, 0)),
        ],
        out_shape=[
            jax.ShapeDtypeStruct((NPAD, 1), jnp.float32),
            jax.ShapeDtypeStruct((NPAD, DH), jnp.float32),
            jax.ShapeDtypeStruct((NPAD, DH), jnp.float32),
        ],
    )(hist, x_pad)


def _tc_agg_h(scat_ref, hs_lo_ref, hs_hi_ref, invs_ref, w_ref, b_ref):
    invs = invs_ref[...]
    agg_lo = invs * (scat_ref[0, 0] + scat_ref[0, 1] + hs_lo_ref[...])
    agg_hi = invs * (scat_ref[1, 0] + scat_ref[1, 1] + hs_hi_ref[...])
    pre = (jnp.dot(agg_lo, w_ref[:DH, :], preferred_element_type=jnp.float32)
           + jnp.dot(agg_hi, w_ref[DH:, :], preferred_element_type=jnp.float32)
           + b_ref[...][None, :])
    return jnp.maximum(pre, 0.0)


def _tc_layer_body(scat_ref, hs_lo_ref, hs_hi_ref, invs_ref, w_ref, b_ref,
                   out_lo_ref, out_hi_ref):
    h = _tc_agg_h(scat_ref, hs_lo_ref, hs_hi_ref, invs_ref, w_ref, b_ref)
    hsn = h * invs_ref[...]
    out_lo_ref[...] = hsn[:, :DH]
    out_hi_ref[...] = hsn[:, DH:]


def _tc_layer(scat, hs_lo, hs_hi, invs, w, b):
    return pl.pallas_call(
        _tc_layer_body,
        grid=(NPAD // _R,),
        in_specs=[
            pl.BlockSpec((2, NC, _R, DH), lambda i: (0, 0, i, 0)),
            pl.BlockSpec((_R, DH), lambda i: (i, 0)),
            pl.BlockSpec((_R, DH), lambda i: (i, 0)),
            pl.BlockSpec((_R, 1), lambda i: (i, 0)),
            pl.BlockSpec((D, HID), lambda i: (0, 0)),
            pl.BlockSpec((HID,), lambda i: (0,)),
        ],
        out_specs=[
            pl.BlockSpec((_R, DH), lambda i: (i, 0)),
            pl.BlockSpec((_R, DH), lambda i: (i, 0)),
        ],
        out_shape=[
            jax.ShapeDtypeStruct((NPAD, DH), jnp.float32),
            jax.ShapeDtypeStruct((NPAD, DH), jnp.float32),
        ],
    )(scat, hs_lo, hs_hi, invs, w, b)


def _tc_final_body(scat_ref, hs_lo_ref, hs_hi_ref, invs_ref, sstage_ref,
                   mask_ref, w2_ref, b2_ref, w3_ref, b3_ref, out_ref, acc_ref):
    i = pl.program_id(0)
    h3 = _tc_agg_h(scat_ref, hs_lo_ref, hs_hi_ref, invs_ref, w2_ref, b2_ref)
    invs = invs_ref[...]
    s = jnp.sum(sstage_ref[...], axis=0)[:, None]
    w = mask_ref[...] * invs * (invs + s)
    contrib = jnp.sum(w * h3, axis=0, keepdims=True)

    @pl.when(i == 0)
    def _():
        acc_ref[...] = contrib

    @pl.when(i > 0)
    def _():
        acc_ref[...] = acc_ref[...] + contrib

    @pl.when(i == NPAD // _R - 1)
    def _():
        pooled = acc_ref[...] * (1.0 / N)
        out_ref[...] = (
            jnp.dot(pooled, w3_ref[...], preferred_element_type=jnp.float32)
            + b3_ref[...][None, :])


def _tc_final(scat, hs_lo, hs_hi, invs, sstage, mask, w2, b2, w3, b3):
    return pl.pallas_call(
        _tc_final_body,
        grid=(NPAD // _R,),
        in_specs=[
            pl.BlockSpec((2, NC, _R, DH), lambda i: (0, 0, i, 0)),
            pl.BlockSpec((_R, DH), lambda i: (i, 0)),
            pl.BlockSpec((_R, DH), lambda i: (i, 0)),
            pl.BlockSpec((_R, 1), lambda i: (i, 0)),
            pl.BlockSpec((NW, _R), lambda i: (0, i)),
            pl.BlockSpec((_R, 1), lambda i: (i, 0)),
            pl.BlockSpec((HID, HID), lambda i: (0, 0)),
            pl.BlockSpec((HID,), lambda i: (0,)),
            pl.BlockSpec((HID, C), lambda i: (0, 0)),
            pl.BlockSpec((C,), lambda i: (0,)),
        ],
        out_specs=pl.BlockSpec((1, C), lambda i: (0, 0)),
        out_shape=jax.ShapeDtypeStruct((1, C), jnp.float32),
        scratch_shapes=[pltpu.VMEM((1, HID), jnp.float32)],
    )(scat, hs_lo, hs_hi, invs, sstage, mask, w2, b2, w3, b3)


def kernel(X, edge_list, W0, b0, W1, b1, W2, b2, W3, b3):
    src_flat = edge_list[0]
    dst_flat = edge_list[1]
    src2d = src_flat.reshape(NW * NCH, CW)
    dst2d = dst_flat.reshape(NW * NCH, CW)
    x_pad = jnp.zeros((NPAD, D), jnp.float32).at[:N].set(X)
    mask = (jnp.arange(NPAD) < N).astype(jnp.float32)[:, None]

    hist = _sc_hist()(dst_flat)
    invs, hs_lo, hs_hi = _tc_prep(hist, x_pad)
    sstage = _sc_s()(src_flat, dst_flat, invs.reshape(NPAD))

    scat0 = _sc_agg()(src2d, dst2d, hs_lo, hs_hi)
    hs_lo, hs_hi = _tc_layer(scat0, hs_lo, hs_hi, invs, W0, b0)
    scat1 = _sc_agg()(src2d, dst2d, hs_lo, hs_hi)
    hs_lo, hs_hi = _tc_layer(scat1, hs_lo, hs_hi, invs, W1, b1)
    scat2 = _sc_agg()(src2d, dst2d, hs_lo, hs_hi)
    return _tc_final(scat2, hs_lo, hs_hi, invs, sstage, mask, W2, b2, W3, b3)


# speedup vs baseline: 22.4787x; 1.1514x over previous
import functools

import jax
import jax.numpy as jnp
from jax import lax
from jax.experimental import pallas as pl
from jax.experimental.pallas import tpu as pltpu
from jax.experimental.pallas import tpu_sc as plsc

NC = 2
NS = 16
NW = NC * NS
LANES = 16

N = 10000
NPAD = 10240
E = 320000
D = 128
DH = D // 2
HID = 128
C = 40

CW = 125
NCHT = E // (NS * CW)
ECHT = NCHT * CW
NCHH = NCHT // 2
NB = 4
WAVES_H = NCHH // NB
RPT = NPAD // NS
ZR = 16
HIST_EPT = E // NW


@functools.cache
def _mesh():
    return plsc.VectorSubcoreMesh(
        core_axis_name="c", subcore_axis_name="s",
        num_cores=NC, num_subcores=NS)


def _sc_hist_body(dst_hbm, out_hbm, idx_v, hist_v):
    wid = lax.axis_index("s") * NC + lax.axis_index("c")
    pltpu.sync_copy(dst_hbm.at[pl.ds(wid * HIST_EPT, HIST_EPT)], idx_v)
    zeros = jnp.zeros((LANES,), jnp.float32)

    def zbody(i, c):
        hist_v[pl.ds(i * LANES, LANES)] = zeros
        return c

    lax.fori_loop(0, NPAD // LANES, zbody, 0)
    ones = jnp.ones((LANES,), jnp.float32)

    def body(i, c):
        idx = idx_v[pl.ds(i * LANES, LANES)]
        plsc.addupdate_scatter(hist_v, [idx], ones)
        return c

    lax.fori_loop(0, HIST_EPT // LANES, body, 0)
    pltpu.sync_copy(hist_v, out_hbm.at[wid])


@functools.cache
def _sc_hist():
    return pl.kernel(
        _sc_hist_body,
        out_type=jax.ShapeDtypeStruct((NW, NPAD), jnp.float32),
        mesh=_mesh(),
        scratch_types=[
            pltpu.VMEM((HIST_EPT,), jnp.int32),
            pltpu.VMEM((NPAD,), jnp.float32),
        ],
        compiler_params=pltpu.CompilerParams(needs_layout_passes=False),
    )


def _sc_s_body(src_hbm, dst_hbm, invs_hbm, out_hbm, src_v, dst_v, invs_v, s_v):
    wid = lax.axis_index("s") * NC + lax.axis_index("c")
    pltpu.sync_copy(src_hbm.at[pl.ds(wid * HIST_EPT, HIST_EPT)], src_v)
    pltpu.sync_copy(dst_hbm.at[pl.ds(wid * HIST_EPT, HIST_EPT)], dst_v)
    pltpu.sync_copy(invs_hbm, invs_v)
    zeros = jnp.zeros((LANES,), jnp.float32)

    def zbody(i, c):
        s_v[pl.ds(i * LANES, LANES)] = zeros
        return c

    lax.fori_loop(0, NPAD // LANES, zbody, 0)

    def body(i, c):
        d16 = dst_v[pl.ds(i * LANES, LANES)]
        s16 = src_v[pl.ds(i * LANES, LANES)]
        vals = plsc.load_gather(invs_v, [d16])
        plsc.addupdate_scatter(s_v, [s16], vals)
        return c

    lax.fori_loop(0, HIST_EPT // LANES, body, 0)
    pltpu.sync_copy(s_v, out_hbm.at[wid])


@functools.cache
def _sc_s():
    return pl.kernel(
        _sc_s_body,
        out_type=jax.ShapeDtypeStruct((NW, NPAD), jnp.float32),
        mesh=_mesh(),
        scratch_types=[
            pltpu.VMEM((HIST_EPT,), jnp.int32),
            pltpu.VMEM((HIST_EPT,), jnp.int32),
            pltpu.VMEM((NPAD,), jnp.float32),
            pltpu.VMEM((NPAD,), jnp.float32),
        ],
        compiler_params=pltpu.CompilerParams(needs_layout_passes=False),
    )


def _sc_agg_body(src_hbm, dst_hbm, hs_hbm, out_hbm, *rest):
    bufs = rest[:NB]
    src_v, dst_v, zbuf, agg_sh = rest[NB:NB + 4]
    gsem = rest[NB + 4:2 * NB + 4]
    ssem = rest[2 * NB + 4:3 * NB + 4]

    cid = lax.axis_index("c")
    sid = lax.axis_index("s")

    zeros = jnp.zeros((LANES,), jnp.float32)

    def zb(i, c):
        r = i // (DH // LANES)
        col = (i % (DH // LANES)) * LANES
        zbuf[r, pl.ds(col, LANES)] = zeros
        return c

    lax.fori_loop(0, ZR * DH // LANES, zb, 0)

    def zcopy(t, c):
        pltpu.sync_copy(zbuf, agg_sh.at[pl.ds(sid * RPT + t * ZR, ZR)])
        return c

    lax.fori_loop(0, RPT // ZR, zcopy, 0)
    plsc.subcore_barrier()

    hsv = hs_hbm.at[cid]

    for h in range(2):
        base = sid * NCHT + h * NCHH
        pltpu.sync_copy(src_hbm.at[pl.ds(base, NCHH)], src_v)
        pltpu.sync_copy(dst_hbm.at[pl.ds(base, NCHH)], dst_v)

        for c in range(NB):
            pltpu.async_copy(hsv.at[src_v.at[c]], bufs[c], gsem[c])

        def wave(i, carry):
            @pl.when(i > 0)
            def _():
                for c in range(NB):
                    j = i * NB + c
                    pltpu.make_async_copy(
                        bufs[c], agg_sh.at[dst_v.at[j - NB]], ssem[c]).wait()
                    pltpu.async_copy(hsv.at[src_v.at[j]], bufs[c], gsem[c])

            for c in range(NB):
                j = i * NB + c
                pltpu.make_async_copy(
                    hsv.at[src_v.at[j]], bufs[c], gsem[c]).wait()
                pltpu.async_copy(
                    bufs[c], agg_sh.at[dst_v.at[j]], ssem[c], add=True)
            return carry

        lax.fori_loop(0, WAVES_H, wave, 0)

        for c in range(NB):
            j = (WAVES_H - 1) * NB + c
            pltpu.make_async_copy(
                bufs[c], agg_sh.at[dst_v.at[j]], ssem[c]).wait()

    plsc.subcore_barrier()
    pltpu.sync_copy(agg_sh.at[pl.ds(sid * RPT, RPT)],
                    out_hbm.at[cid].at[pl.ds(sid * RPT, RPT)])


@functools.cache
def _sc_agg():
    scratch = [pltpu.VMEM((CW, DH), jnp.float32) for _ in range(NB)]
    scratch += [
        pltpu.VMEM((NCHH, CW), jnp.int32),
        pltpu.VMEM((NCHH, CW), jnp.int32),
        pltpu.VMEM((ZR, DH), jnp.float32),
        pltpu.VMEM_SHARED((NPAD, DH), jnp.float32),
    ]
    scratch += [pltpu.SemaphoreType.DMA for _ in range(2 * NB)]
    return pl.kernel(
        _sc_agg_body,
        out_type=jax.ShapeDtypeStruct((NC, NPAD, DH), jnp.float32),
        mesh=_mesh(),
        scratch_types=scratch,
        compiler_params=pltpu.CompilerParams(
            needs_layout_passes=False, use_tc_tiling_on_sc=False),
    )


_R = 1024


def _tc_prep_body(hist_ref, x_ref, invs_ref, hs_ref):
    deg = 1.0 + jnp.sum(hist_ref[...], axis=0)
    invs = lax.rsqrt(deg)
    invs_ref[...] = invs[:, None]
    hs = x_ref[...] * invs[:, None]
    hs_ref[0] = hs[:, :DH]
    hs_ref[1] = hs[:, DH:]


def _tc_prep(hist, x_pad):
    return pl.pallas_call(
        _tc_prep_body,
        grid=(NPAD // _R,),
        in_specs=[
            pl.BlockSpec((NW, _R), lambda i: (0, i)),
            pl.BlockSpec((_R, D), lambda i: (i, 0)),
        ],
        out_specs=[
            pl.BlockSpec((_R, 1), lambda i: (i, 0)),
            pl.BlockSpec((NC, _R, DH), lambda i: (0, i, 0)),
        ],
        out_shape=[
            jax.ShapeDtypeStruct((NPAD, 1), jnp.float32),
            jax.ShapeDtypeStruct((NC, NPAD, DH), jnp.float32),
        ],
    )(hist, x_pad)


def _tc_agg_h(scat_ref, hs_ref, invs_ref, w_ref, b_ref):
    invs = invs_ref[...]
    agg_lo = invs * (scat_ref[0] + hs_ref[0])
    agg_hi = invs * (scat_ref[1] + hs_ref[1])
    pre = (jnp.dot(agg_lo, w_ref[:DH, :], preferred_element_type=jnp.float32)
           + jnp.dot(agg_hi, w_ref[DH:, :], preferred_element_type=jnp.float32)
           + b_ref[...][None, :])
    return jnp.maximum(pre, 0.0)


def _tc_layer_body(scat_ref, hs_ref, invs_ref, w_ref, b_ref, out_ref):
    h = _tc_agg_h(scat_ref, hs_ref, invs_ref, w_ref, b_ref)
    hsn = h * invs_ref[...]
    out_ref[0] = hsn[:, :DH]
    out_ref[1] = hsn[:, DH:]


def _tc_layer(scat, hs, invs, w, b):
    return pl.pallas_call(
        _tc_layer_body,
        grid=(NPAD // _R,),
        in_specs=[
            pl.BlockSpec((NC, _R, DH), lambda i: (0, i, 0)),
            pl.BlockSpec((NC, _R, DH), lambda i: (0, i, 0)),
            pl.BlockSpec((_R, 1), lambda i: (i, 0)),
            pl.BlockSpec((D, HID), lambda i: (0, 0)),
            pl.BlockSpec((HID,), lambda i: (0,)),
        ],
        out_specs=pl.BlockSpec((NC, _R, DH), lambda i: (0, i, 0)),
        out_shape=jax.ShapeDtypeStruct((NC, NPAD, DH), jnp.float32),
    )(scat, hs, invs, w, b)


def _tc_final_body(scat_ref, hs_ref, invs_ref, sstage_ref, mask_ref,
                   w2_ref, b2_ref, w3_ref, b3_ref, out_ref, acc_ref):
    i = pl.program_id(0)
    h3 = _tc_agg_h(scat_ref, hs_ref, invs_ref, w2_ref, b2_ref)
    invs = invs_ref[...]
    s = jnp.sum(sstage_ref[...], axis=0)[:, None]
    w = mask_ref[...] * invs * (invs + s)
    contrib = jnp.sum(w * h3, axis=0, keepdims=True)

    @pl.when(i == 0)
    def _():
        acc_ref[...] = contrib

    @pl.when(i > 0)
    def _():
        acc_ref[...] = acc_ref[...] + contrib

    @pl.when(i == NPAD // _R - 1)
    def _():
        pooled = acc_ref[...] * (1.0 / N)
        out_ref[...] = (
            jnp.dot(pooled, w3_ref[...], preferred_element_type=jnp.float32)
            + b3_ref[...][None, :])


def _tc_final(scat, hs, invs, sstage, mask, w2, b2, w3, b3):
    return pl.pallas_call(
        _tc_final_body,
        grid=(NPAD // _R,),
        in_specs=[
            pl.BlockSpec((NC, _R, DH), lambda i: (0, i, 0)),
            pl.BlockSpec((NC, _R, DH), lambda i: (0, i, 0)),
            pl.BlockSpec((_R, 1), lambda i: (i, 0)),
            pl.BlockSpec((NW, _R), lambda i: (0, i)),
            pl.BlockSpec((_R, 1), lambda i: (i, 0)),
            pl.BlockSpec((HID, HID), lambda i: (0, 0)),
            pl.BlockSpec((HID,), lambda i: (0,)),
            pl.BlockSpec((HID, C), lambda i: (0, 0)),
            pl.BlockSpec((C,), lambda i: (0,)),
        ],
        out_specs=pl.BlockSpec((1, C), lambda i: (0, 0)),
        out_shape=jax.ShapeDtypeStruct((1, C), jnp.float32),
        scratch_shapes=[pltpu.VMEM((1, HID), jnp.float32)],
    )(scat, hs, invs, sstage, mask, w2, b2, w3, b3)


def kernel(X, edge_list, W0, b0, W1, b1, W2, b2, W3, b3):
    src_flat = edge_list[0]
    dst_flat = edge_list[1]
    src2d = src_flat.reshape(NS * NCHT, CW)
    dst2d = dst_flat.reshape(NS * NCHT, CW)
    x_pad = jnp.zeros((NPAD, D), jnp.float32).at[:N].set(X)
    mask = (jnp.arange(NPAD) < N).astype(jnp.float32)[:, None]

    hist = _sc_hist()(dst_flat)
    invs, hs = _tc_prep(hist, x_pad)

    sstage = _sc_s()(src_flat, dst_flat, invs.reshape(NPAD))
    scat0 = _sc_agg()(src2d, dst2d, hs)
    hs = _tc_layer(scat0, hs, invs, W0, b0)
    scat1 = _sc_agg()(src2d, dst2d, hs)
    hs = _tc_layer(scat1, hs, invs, W1, b1)
    scat2 = _sc_agg()(src2d, dst2d, hs)
    return _tc_final(scat2, hs, invs, sstage, mask, W2, b2, W3, b3)


# speedup vs baseline: 22.4985x; 1.0009x over previous
import functools

import jax
import jax.numpy as jnp
from jax import lax
from jax.experimental import pallas as pl
from jax.experimental.pallas import tpu as pltpu
from jax.experimental.pallas import tpu_sc as plsc

NC = 2
NS = 16
NW = NC * NS
LANES = 16

N = 10000
NPAD = 10240
E = 320000
D = 128
DH = D // 2
HID = 128
C = 40

CW = 125
NCHT = E // (NS * CW)
ECHT = NCHT * CW
NCHH = NCHT // 2
NB = 4
WAVES_H = NCHH // NB
RPT = NPAD // NS
ZR = 128
HIST_EPT = E // NW


@functools.cache
def _mesh():
    return plsc.VectorSubcoreMesh(
        core_axis_name="c", subcore_axis_name="s",
        num_cores=NC, num_subcores=NS)


def _sc_hist_body(dst_hbm, out_hbm, idx_v, hist_v):
    wid = lax.axis_index("s") * NC + lax.axis_index("c")
    pltpu.sync_copy(dst_hbm.at[pl.ds(wid * HIST_EPT, HIST_EPT)], idx_v)
    zeros = jnp.zeros((LANES,), jnp.float32)

    def zbody(i, c):
        hist_v[pl.ds(i * LANES, LANES)] = zeros
        return c

    lax.fori_loop(0, NPAD // LANES, zbody, 0)
    ones = jnp.ones((LANES,), jnp.float32)

    def body(i, c):
        idx = idx_v[pl.ds(i * LANES, LANES)]
        plsc.addupdate_scatter(hist_v, [idx], ones)
        return c

    lax.fori_loop(0, HIST_EPT // LANES, body, 0)
    pltpu.sync_copy(hist_v, out_hbm.at[wid])


@functools.cache
def _sc_hist():
    return pl.kernel(
        _sc_hist_body,
        out_type=jax.ShapeDtypeStruct((NW, NPAD), jnp.float32),
        mesh=_mesh(),
        scratch_types=[
            pltpu.VMEM((HIST_EPT,), jnp.int32),
            pltpu.VMEM((NPAD,), jnp.float32),
        ],
        compiler_params=pltpu.CompilerParams(needs_layout_passes=False),
    )


def _sc_s_body(src_hbm, dst_hbm, invs_hbm, out_hbm, src_v, dst_v, invs_v, s_v):
    wid = lax.axis_index("s") * NC + lax.axis_index("c")
    pltpu.sync_copy(src_hbm.at[pl.ds(wid * HIST_EPT, HIST_EPT)], src_v)
    pltpu.sync_copy(dst_hbm.at[pl.ds(wid * HIST_EPT, HIST_EPT)], dst_v)
    pltpu.sync_copy(invs_hbm, invs_v)
    zeros = jnp.zeros((LANES,), jnp.float32)

    def zbody(i, c):
        s_v[pl.ds(i * LANES, LANES)] = zeros
        return c

    lax.fori_loop(0, NPAD // LANES, zbody, 0)

    def body(i, c):
        d16 = dst_v[pl.ds(i * LANES, LANES)]
        s16 = src_v[pl.ds(i * LANES, LANES)]
        vals = plsc.load_gather(invs_v, [d16])
        plsc.addupdate_scatter(s_v, [s16], vals)
        return c

    lax.fori_loop(0, HIST_EPT // LANES, body, 0)
    pltpu.sync_copy(s_v, out_hbm.at[wid])


@functools.cache
def _sc_s():
    return pl.kernel(
        _sc_s_body,
        out_type=jax.ShapeDtypeStruct((NW, NPAD), jnp.float32),
        mesh=_mesh(),
        scratch_types=[
            pltpu.VMEM((HIST_EPT,), jnp.int32),
            pltpu.VMEM((HIST_EPT,), jnp.int32),
            pltpu.VMEM((NPAD,), jnp.float32),
            pltpu.VMEM((NPAD,), jnp.float32),
        ],
        compiler_params=pltpu.CompilerParams(needs_layout_passes=False),
    )


def _sc_agg_body(src_hbm, dst_hbm, hs_hbm, out_hbm, *rest):
    bufs = rest[:NB]
    src_v, dst_v, zbuf, agg_sh = rest[NB:NB + 4]
    gsem = rest[NB + 4:2 * NB + 4]
    ssem = rest[2 * NB + 4:3 * NB + 4]

    cid = lax.axis_index("c")
    sid = lax.axis_index("s")

    zeros = jnp.zeros((LANES,), jnp.float32)

    def zb(i, c):
        r = i // (DH // LANES)
        col = (i % (DH // LANES)) * LANES
        zbuf[r, pl.ds(col, LANES)] = zeros
        return c

    lax.fori_loop(0, ZR * DH // LANES, zb, 0)

    for t in range(RPT // ZR):
        pltpu.async_copy(zbuf, agg_sh.at[pl.ds(sid * RPT + t * ZR, ZR)],
                         gsem[t % NB])
    for t in range(RPT // ZR):
        pltpu.make_async_copy(zbuf, agg_sh.at[pl.ds(sid * RPT + t * ZR, ZR)],
                              gsem[t % NB]).wait()
    plsc.subcore_barrier()

    hsv = hs_hbm.at[cid]

    for h in range(2):
        base = sid * NCHT + h * NCHH
        pltpu.sync_copy(src_hbm.at[pl.ds(base, NCHH)], src_v)
        pltpu.sync_copy(dst_hbm.at[pl.ds(base, NCHH)], dst_v)

        for c in range(NB):
            pltpu.async_copy(hsv.at[src_v.at[c]], bufs[c], gsem[c])

        def wave(i, carry):
            @pl.when(i > 0)
            def _():
                for c in range(NB):
                    j = i * NB + c
                    pltpu.make_async_copy(
                        bufs[c], agg_sh.at[dst_v.at[j - NB]], ssem[c]).wait()
                    pltpu.async_copy(hsv.at[src_v.at[j]], bufs[c], gsem[c])

            for c in range(NB):
                j = i * NB + c
                pltpu.make_async_copy(
                    hsv.at[src_v.at[j]], bufs[c], gsem[c]).wait()
                pltpu.async_copy(
                    bufs[c], agg_sh.at[dst_v.at[j]], ssem[c], add=True)
            return carry

        lax.fori_loop(0, WAVES_H, wave, 0)

        for c in range(NB):
            j = (WAVES_H - 1) * NB + c
            pltpu.make_async_copy(
                bufs[c], agg_sh.at[dst_v.at[j]], ssem[c]).wait()

    plsc.subcore_barrier()
    pltpu.sync_copy(agg_sh.at[pl.ds(sid * RPT, RPT)],
                    out_hbm.at[cid].at[pl.ds(sid * RPT, RPT)])


@functools.cache
def _sc_agg():
    scratch = [pltpu.VMEM((CW, DH), jnp.float32) for _ in range(NB)]
    scratch += [
        pltpu.VMEM((NCHH, CW), jnp.int32),
        pltpu.VMEM((NCHH, CW), jnp.int32),
        pltpu.VMEM((ZR, DH), jnp.float32),
        pltpu.VMEM_SHARED((NPAD, DH), jnp.float32),
    ]
    scratch += [pltpu.SemaphoreType.DMA for _ in range(2 * NB)]
    return pl.kernel(
        _sc_agg_body,
        out_type=jax.ShapeDtypeStruct((NC, NPAD, DH), jnp.float32),
        mesh=_mesh(),
        scratch_types=scratch,
        compiler_params=pltpu.CompilerParams(
            needs_layout_passes=False, use_tc_tiling_on_sc=False),
    )


_R = 1024


def _tc_prep_body(hist_ref, x_ref, invs_ref, hs_ref):
    deg = 1.0 + jnp.sum(hist_ref[...], axis=0)
    invs = lax.rsqrt(deg)
    invs_ref[...] = invs[:, None]
    hs = x_ref[...] * invs[:, None]
    hs_ref[0] = hs[:, :DH]
    hs_ref[1] = hs[:, DH:]


def _tc_prep(hist, x_pad):
    return pl.pallas_call(
        _tc_prep_body,
        grid=(NPAD // _R,),
        in_specs=[
            pl.BlockSpec((NW, _R), lambda i: (0, i)),
            pl.BlockSpec((_R, D), lambda i: (i, 0)),
        ],
        out_specs=[
            pl.BlockSpec((_R, 1), lambda i: (i, 0)),
            pl.BlockSpec((NC, _R, DH), lambda i: (0, i, 0)),
        ],
        out_shape=[
            jax.ShapeDtypeStruct((NPAD, 1), jnp.float32),
            jax.ShapeDtypeStruct((NC, NPAD, DH), jnp.float32),
        ],
    )(hist, x_pad)


def _tc_agg_h(scat_ref, hs_ref, invs_ref, w_ref, b_ref):
    invs = invs_ref[...]
    agg_lo = invs * (scat_ref[0] + hs_ref[0])
    agg_hi = invs * (scat_ref[1] + hs_ref[1])
    pre = (jnp.dot(agg_lo, w_ref[:DH, :], preferred_element_type=jnp.float32)
           + jnp.dot(agg_hi, w_ref[DH:, :], preferred_element_type=jnp.float32)
           + b_ref[...][None, :])
    return jnp.maximum(pre, 0.0)


def _tc_layer_body(scat_ref, hs_ref, invs_ref, w_ref, b_ref, out_ref):
    h = _tc_agg_h(scat_ref, hs_ref, invs_ref, w_ref, b_ref)
    hsn = h * invs_ref[...]
    out_ref[0] = hsn[:, :DH]
    out_ref[1] = hsn[:, DH:]


def _tc_layer(scat, hs, invs, w, b):
    return pl.pallas_call(
        _tc_layer_body,
        grid=(NPAD // _R,),
        in_specs=[
            pl.BlockSpec((NC, _R, DH), lambda i: (0, i, 0)),
            pl.BlockSpec((NC, _R, DH), lambda i: (0, i, 0)),
            pl.BlockSpec((_R, 1), lambda i: (i, 0)),
            pl.BlockSpec((D, HID), lambda i: (0, 0)),
            pl.BlockSpec((HID,), lambda i: (0,)),
        ],
        out_specs=pl.BlockSpec((NC, _R, DH), lambda i: (0, i, 0)),
        out_shape=jax.ShapeDtypeStruct((NC, NPAD, DH), jnp.float32),
    )(scat, hs, invs, w, b)


def _tc_final_body(scat_ref, hs_ref, invs_ref, sstage_ref, mask_ref,
                   w2_ref, b2_ref, w3_ref, b3_ref, out_ref, acc_ref):
    i = pl.program_id(0)
    h3 = _tc_agg_h(scat_ref, hs_ref, invs_ref, w2_ref, b2_ref)
    invs = invs_ref[...]
    s = jnp.sum(sstage_ref[...], axis=0)[:, None]
    w = mask_ref[...] * invs * (invs + s)
    contrib = jnp.sum(w * h3, axis=0, keepdims=True)

    @pl.when(i == 0)
    def _():
        acc_ref[...] = contrib

    @pl.when(i > 0)
    def _():
        acc_ref[...] = acc_ref[...] + contrib

    @pl.when(i == NPAD // _R - 1)
    def _():
        pooled = acc_ref[...] * (1.0 / N)
        out_ref[...] = (
            jnp.dot(pooled, w3_ref[...], preferred_element_type=jnp.float32)
            + b3_ref[...][None, :])


def _tc_final(scat, hs, invs, sstage, mask, w2, b2, w3, b3):
    return pl.pallas_call(
        _tc_final_body,
        grid=(NPAD // _R,),
        in_specs=[
            pl.BlockSpec((NC, _R, DH), lambda i: (0, i, 0)),
            pl.BlockSpec((NC, _R, DH), lambda i: (0, i, 0)),
            pl.BlockSpec((_R, 1), lambda i: (i, 0)),
            pl.BlockSpec((NW, _R), lambda i: (0, i)),
            pl.BlockSpec((_R, 1), lambda i: (i, 0)),
            pl.BlockSpec((HID, HID), lambda i: (0, 0)),
            pl.BlockSpec((HID,), lambda i: (0,)),
            pl.BlockSpec((HID, C), lambda i: (0, 0)),
            pl.BlockSpec((C,), lambda i: (0,)),
        ],
        out_specs=pl.BlockSpec((1, C), lambda i: (0, 0)),
        out_shape=jax.ShapeDtypeStruct((1, C), jnp.float32),
        scratch_shapes=[pltpu.VMEM((1, HID), jnp.float32)],
    )(scat, hs, invs, sstage, mask, w2, b2, w3, b3)


def kernel(X, edge_list, W0, b0, W1, b1, W2, b2, W3, b3):
    src_flat = edge_list[0]
    dst_flat = edge_list[1]
    src2d = src_flat.reshape(NS * NCHT, CW)
    dst2d = dst_flat.reshape(NS * NCHT, CW)
    x_pad = jnp.zeros((NPAD, D), jnp.float32).at[:N].set(X)
    mask = (jnp.arange(NPAD) < N).astype(jnp.float32)[:, None]

    hist = _sc_hist()(dst_flat)
    invs, hs = _tc_prep(hist, x_pad)

    scat0 = _sc_agg()(src2d, dst2d, hs)
    sstage = _sc_s()(src_flat, dst_flat, invs.reshape(NPAD))
    hs = _tc_layer(scat0, hs, invs, W0, b0)
    scat1 = _sc_agg()(src2d, dst2d, hs)
    hs = _tc_layer(scat1, hs, invs, W1, b1)
    scat2 = _sc_agg()(src2d, dst2d, hs)
    return _tc_final(scat2, hs, invs, sstage, mask, W2, b2, W3, b3)
